# baseline (speedup 1.0000x reference)
import functools

import jax
import jax.numpy as jnp
from jax import lax
from jax.experimental import pallas as pl
from jax.experimental.pallas import tpu as pltpu
from jax.experimental.pallas import tpu_sc as plsc

N = 10000
E = 320000
D = 128
R = 8

NC = 2
NS = 16
NW = NC * NS
CHUNK = 128
NCHUNK = 80
EPT = CHUNK * NCHUNK
ROWS_PT = 624
TAIL_ROWS = N - 16 * ROWS_PT
ZBLK = 104
DH = D // 2
LANES_H = DH // 16
N2 = N // 2


def _mm_body(f_ref, w_ref, o_ref):
    f = f_ref[...]
    w0 = w_ref[0]
    fa = f[:, :D]
    fb = f[:, D:]
    for h in range(2):
        wh = w0[:, h * DH:(h + 1) * DH]
        pa = jnp.dot(fa, wh, preferred_element_type=jnp.float32)
        pb = jnp.dot(fb, wh, preferred_element_type=jnp.float32)
        o_ref[h] = jnp.concatenate([pa, pb], axis=1)


def _relation_transform(f2, weight):
    bm = 1000
    nb = N2 // bm
    return pl.pallas_call(
        _mm_body,
        grid=(nb, R),
        in_specs=[
            pl.BlockSpec((bm, 2 * D), lambda b, r: (b, 0)),
            pl.BlockSpec((1, D, D), lambda b, r: (r, 0, 0)),
        ],
        out_specs=pl.BlockSpec((2, bm, D), lambda b, r, nb=nb: (0, r * nb + b, 0)),
        out_shape=jax.ShapeDtypeStruct((2, R * N2, D), jnp.float32),
    )(f2, weight)


def _add_body(p_ref, o_ref):
    o_ref[...] = jnp.concatenate(
        [p_ref[0, 0] + p_ref[1, 0], p_ref[0, 1] + p_ref[1, 1]], axis=-1
    )


def _combine_partials(partials):
    bn = 2000
    return pl.pallas_call(
        _add_body,
        grid=(N // bn,),
        in_specs=[pl.BlockSpec((NC, 2, bn, DH), lambda b: (0, 0, b, 0))],
        out_specs=pl.BlockSpec((bn, D), lambda b: (b, 0)),
        out_shape=jax.ShapeDtypeStruct((N, D), jnp.float32),
    )(partials)


def _sc_body(hr_hbm, g_hbm, dst_hbm, norm_hbm, out_hbm,
             g_v, d_v, n_v, gbuf0, gbuf1, sbuf0, sbuf1, acc,
             gsem0, gsem1, ssem0, ssem1):
    cid = lax.axis_index("c")
    sid = lax.axis_index("s")
    w = cid * NS + sid

    gbufs = (gbuf0, gbuf1)
    sbufs = (sbuf0, sbuf1)
    gsems = (gsem0, gsem1)
    ssems = (ssem0, ssem1)

    pltpu.sync_copy(g_hbm.at[w], g_v)
    pltpu.sync_copy(dst_hbm.at[w], d_v)
    pltpu.sync_copy(norm_hbm.at[w], n_v)

    zeros16 = jnp.zeros((16,), jnp.float32)

    for h in range(2):
        tab = hr_hbm.at[h]

        @pl.loop(0, ZBLK)
        def _(e):
            for k in range(LANES_H):
                gbuf0[e, pl.ds(k * 16, 16)] = zeros16

        for j in range(ROWS_PT // ZBLK):
            pltpu.sync_copy(gbuf0.at[pl.ds(0, ZBLK)],
                            acc.at[pl.ds(sid * ROWS_PT + j * ZBLK, ZBLK)])

        @pl.when(sid == NS - 1)
        def _():
            pltpu.sync_copy(gbuf0.at[pl.ds(0, TAIL_ROWS)],
                            acc.at[pl.ds(N - TAIL_ROWS, TAIL_ROWS)])

        def start_gather(c, b):
            pltpu.async_copy(tab.at[g_v.at[c]], gbufs[b], gsems[b])

        def wait_gather(c, b):
            pltpu.make_async_copy(tab.at[g_v.at[c]], gbufs[b], gsems[b]).wait()

        def start_scatter(c, b):
            pltpu.async_copy(sbufs[b], acc.at[d_v.at[c]], ssems[b], add=True)

        def wait_scatter(c, b):
            pltpu.make_async_copy(sbufs[b], acc.at[d_v.at[c]], ssems[b]).wait()

        plsc.subcore_barrier()

        start_gather(0, 0)
        start_gather(1, 1)

        @pl.loop(0, NCHUNK, step=2)
        def _(c0):
            for b in range(2):
                c = c0 + b
                wait_gather(c, b)

                @pl.when(c0 > 0)
                def _():
                    wait_scatter(c - 2, b)

                def scale_group(off):
                    nv = n_v[c, pl.ds(off, 16)]
                    for i in range(16):
                        s = nv[i]
                        for k in range(LANES_H):
                            sl = pl.ds(k * 16, 16)
                            sbufs[b][off + i, sl] = gbufs[b][off + i, sl] * s

                @pl.loop(0, CHUNK // 16)
                def _(eg):
                    scale_group(eg * 16)

                @pl.when(c0 < NCHUNK - 2)
                def _():
                    start_gather(c + 2, b)

                start_scatter(c, b)

        wait_scatter(NCHUNK - 2, 0)
        wait_scatter(NCHUNK - 1, 1)

        plsc.subcore_barrier()
        pltpu.sync_copy(acc.at[pl.ds(sid * ROWS_PT, ROWS_PT)],
                        out_hbm.at[cid, h, pl.ds(sid * ROWS_PT, ROWS_PT)])

        @pl.when(sid == NS - 1)
        def _():
            pltpu.sync_copy(acc.at[pl.ds(N - TAIL_ROWS, TAIL_ROWS)],
                            out_hbm.at[cid, h, pl.ds(N - TAIL_ROWS, TAIL_ROWS)])


@functools.partial(
    pl.kernel,
    out_type=jax.ShapeDtypeStruct((NC, 2, N, DH), jnp.float32),
    mesh=plsc.VectorSubcoreMesh(
        core_axis_name="c", subcore_axis_name="s", num_cores=NC, num_subcores=NS
    ),
    compiler_params=pltpu.CompilerParams(use_tc_tiling_on_sc=False),
    scratch_types=[
        pltpu.VMEM((NCHUNK, CHUNK), jnp.int32),
        pltpu.VMEM((NCHUNK, CHUNK), jnp.int32),
        pltpu.VMEM((NCHUNK, CHUNK), jnp.float32),
        pltpu.VMEM((CHUNK, DH), jnp.float32),
        pltpu.VMEM((CHUNK, DH), jnp.float32),
        pltpu.VMEM((CHUNK, DH), jnp.float32),
        pltpu.VMEM((CHUNK, DH), jnp.float32),
        pltpu.VMEM_SHARED((N, DH), jnp.float32),
        pltpu.SemaphoreType.DMA,
        pltpu.SemaphoreType.DMA,
        pltpu.SemaphoreType.DMA,
        pltpu.SemaphoreType.DMA,
    ],
)
def _sc_edge_kernel(hr_hbm, g_hbm, dst_hbm, norm_hbm, out_hbm, *rest):
    _sc_body(hr_hbm, g_hbm, dst_hbm, norm_hbm, out_hbm, *rest)


def kernel(feature, edge_index, edge_type, norm, weight):
    f2 = feature.reshape(N2, 2 * D)
    hr = _relation_transform(f2, weight)
    src = edge_index[0]
    dst = edge_index[1]
    g = edge_type.astype(jnp.int32) * N + src
    pad = NW * EPT - E
    gp = jnp.concatenate([g, jnp.zeros((pad,), jnp.int32)])
    dp = jnp.concatenate([dst, jnp.zeros((pad,), jnp.int32)])
    npad = jnp.concatenate([norm.reshape(E), jnp.zeros((pad,), jnp.float32)])
    partials = _sc_edge_kernel(
        hr.reshape(2, 2 * R * N2, DH),
        gp.reshape(NW, NCHUNK, CHUNK),
        dp.reshape(NW, NCHUNK, CHUNK),
        npad.reshape(NW, NCHUNK, CHUNK),
    )
    return _combine_partials(partials)


# speedup vs baseline: 2.3188x; 2.3188x over previous
import functools

import jax
import jax.numpy as jnp
from jax import lax
from jax.experimental import pallas as pl
from jax.experimental.pallas import tpu as pltpu
from jax.experimental.pallas import tpu_sc as plsc

N = 10000
E = 320000
D = 128
R = 8

NC = 2
NS = 16
NW = NC * NS
CHUNK = 128
NCHUNK = 80
EPT = CHUNK * NCHUNK
ROWS_PT = 624
TAIL_ROWS = N - 16 * ROWS_PT
ZBLK = 104
DH = D // 2
LANES_H = DH // 16
N2 = N // 2


def _mm_body(f_ref, w_ref, o_ref):
    f = f_ref[...]
    w0 = w_ref[0]
    fa = f[:, :D]
    fb = f[:, D:]
    for h in range(2):
        wh = w0[:, h * DH:(h + 1) * DH]
        pa = jnp.dot(fa, wh, preferred_element_type=jnp.float32)
        pb = jnp.dot(fb, wh, preferred_element_type=jnp.float32)
        o_ref[h] = jnp.concatenate([pa, pb], axis=1)


def _relation_transform(f2, weight):
    bm = 1000
    nb = N2 // bm
    return pl.pallas_call(
        _mm_body,
        grid=(nb, R),
        in_specs=[
            pl.BlockSpec((bm, 2 * D), lambda b, r: (b, 0)),
            pl.BlockSpec((1, D, D), lambda b, r: (r, 0, 0)),
        ],
        out_specs=pl.BlockSpec((2, bm, D), lambda b, r, nb=nb: (0, r * nb + b, 0)),
        out_shape=jax.ShapeDtypeStruct((2, R * N2, D), jnp.float32),
    )(f2, weight)


def _add_body(p_ref, o_ref):
    o_ref[...] = jnp.concatenate(
        [p_ref[0, 0] + p_ref[1, 0], p_ref[0, 1] + p_ref[1, 1]], axis=-1
    )


def _combine_partials(partials):
    bn = 2000
    return pl.pallas_call(
        _add_body,
        grid=(N // bn,),
        in_specs=[pl.BlockSpec((NC, 2, bn, DH), lambda b: (0, 0, b, 0))],
        out_specs=pl.BlockSpec((bn, D), lambda b: (b, 0)),
        out_shape=jax.ShapeDtypeStruct((N, D), jnp.float32),
    )(partials)


def _sc_body(hr_hbm, g_hbm, dst_hbm, norm_hbm, out_hbm,
             g_v, d_v, n_v, gbuf0, gbuf1, sbuf0, sbuf1, acc,
             gsem0, gsem1, ssem0, ssem1):
    cid = lax.axis_index("c")
    sid = lax.axis_index("s")
    w = cid * NS + sid

    gbufs = (gbuf0, gbuf1)
    sbufs = (sbuf0, sbuf1)
    gsems = (gsem0, gsem1)
    ssems = (ssem0, ssem1)

    pltpu.sync_copy(g_hbm.at[w], g_v)
    pltpu.sync_copy(dst_hbm.at[w], d_v)
    pltpu.sync_copy(norm_hbm.at[w], n_v)

    zeros16 = jnp.zeros((16,), jnp.float32)

    for h in range(2):
        tab = hr_hbm.at[h]

        @pl.loop(0, ZBLK)
        def _(e):
            for k in range(LANES_H):
                gbuf0[e, pl.ds(k * 16, 16)] = zeros16

        for j in range(ROWS_PT // ZBLK):
            pltpu.sync_copy(gbuf0.at[pl.ds(0, ZBLK)],
                            acc.at[pl.ds(sid * ROWS_PT + j * ZBLK, ZBLK)])

        @pl.when(sid == NS - 1)
        def _():
            pltpu.sync_copy(gbuf0.at[pl.ds(0, TAIL_ROWS)],
                            acc.at[pl.ds(N - TAIL_ROWS, TAIL_ROWS)])

        def start_gather(c, b):
            pltpu.async_copy(tab.at[g_v.at[c]], gbufs[b], gsems[b])

        def wait_gather(c, b):
            pltpu.make_async_copy(tab.at[g_v.at[c]], gbufs[b], gsems[b]).wait()

        def start_scatter(c, b):
            pltpu.async_copy(sbufs[b], acc.at[d_v.at[c]], ssems[b], add=True)

        def wait_scatter(c, b):
            pltpu.make_async_copy(sbufs[b], acc.at[d_v.at[c]], ssems[b]).wait()

        plsc.subcore_barrier()

        start_gather(0, 0)
        start_gather(1, 1)

        @pl.loop(0, NCHUNK, step=2)
        def _(c0):
            for b in range(2):
                c = c0 + b
                wait_gather(c, b)

                @pl.when(c0 > 0)
                def _():
                    wait_scatter(c - 2, b)

                def scale_group(off):
                    nv = n_v[c, pl.ds(off, 16)]
                    for i in range(16):
                        s = nv[i]
                        for k in range(LANES_H):
                            sl = pl.ds(k * 16, 16)
                            sbufs[b][off + i, sl] = gbufs[b][off + i, sl] * s

                @pl.loop(0, CHUNK // 16)
                def _(eg):
                    scale_group(eg * 16)

                @pl.when(c0 < NCHUNK - 2)
                def _():
                    start_gather(c + 2, b)

                start_scatter(c, b)

        wait_scatter(NCHUNK - 2, 0)
        wait_scatter(NCHUNK - 1, 1)

        plsc.subcore_barrier()
        pltpu.sync_copy(acc.at[pl.ds(sid * ROWS_PT, ROWS_PT)],
                        out_hbm.at[cid, h, pl.ds(sid * ROWS_PT, ROWS_PT)])

        @pl.when(sid == NS - 1)
        def _():
            pltpu.sync_copy(acc.at[pl.ds(N - TAIL_ROWS, TAIL_ROWS)],
                            out_hbm.at[cid, h, pl.ds(N - TAIL_ROWS, TAIL_ROWS)])


@functools.partial(
    pl.kernel,
    out_type=jax.ShapeDtypeStruct((NC, 2, N, DH), jnp.float32),
    mesh=plsc.VectorSubcoreMesh(
        core_axis_name="c", subcore_axis_name="s", num_cores=NC, num_subcores=NS
    ),
    compiler_params=pltpu.CompilerParams(use_tc_tiling_on_sc=False),
    scratch_types=[
        pltpu.VMEM((NCHUNK, CHUNK), jnp.int32),
        pltpu.VMEM((NCHUNK, CHUNK), jnp.int32),
        pltpu.VMEM((NCHUNK, CHUNK), jnp.float32),
        pltpu.VMEM((CHUNK, DH), jnp.float32),
        pltpu.VMEM((CHUNK, DH), jnp.float32),
        pltpu.VMEM((CHUNK, DH), jnp.float32),
        pltpu.VMEM((CHUNK, DH), jnp.float32),
        pltpu.VMEM_SHARED((N, DH), jnp.float32),
        pltpu.SemaphoreType.DMA,
        pltpu.SemaphoreType.DMA,
        pltpu.SemaphoreType.DMA,
        pltpu.SemaphoreType.DMA,
    ],
)
def _sc_edge_kernel(hr_hbm, g_hbm, dst_hbm, norm_hbm, out_hbm, *rest):
    _sc_body(hr_hbm, g_hbm, dst_hbm, norm_hbm, out_hbm, *rest)


def kernel(feature, edge_index, edge_type, norm, weight):
    f2 = feature.reshape(N2, 2 * D)
    hr = _relation_transform(f2, weight)
    src = edge_index[0]
    dst = edge_index[1]
    g = edge_type.astype(jnp.int32) * N + src
    pad = NW * EPT - E
    pad_idx = jnp.arange(pad, dtype=jnp.int32)
    gp = jnp.concatenate([g, pad_idx % (2 * R * N2)])
    dp = jnp.concatenate([dst, pad_idx % N])
    npad = jnp.concatenate([norm.reshape(E), jnp.zeros((pad,), jnp.float32)])
    partials = _sc_edge_kernel(
        hr.reshape(2, 2 * R * N2, DH),
        gp.reshape(NW, NCHUNK, CHUNK),
        dp.reshape(NW, NCHUNK, CHUNK),
        npad.reshape(NW, NCHUNK, CHUNK),
    )
    return _combine_partials(partials)


# speedup vs baseline: 2.6425x; 1.1396x over previous
import functools

import jax
import jax.numpy as jnp
from jax import lax
from jax.experimental import pallas as pl
from jax.experimental.pallas import tpu as pltpu
from jax.experimental.pallas import tpu_sc as plsc

N = 10000
E = 320000
D = 128
R = 8

NC = 2
NS = 16
NW = NC * NS
CHUNK = 128
NCHUNK = 80
EPT = CHUNK * NCHUNK
ROWS_PT = 624
TAIL_ROWS = N - 16 * ROWS_PT
ZBLK = 104
DH = D // 2
LANES_H = DH // 16
N2 = N // 2


def _mm_body(f_ref, w_ref, o_ref):
    f = f_ref[...]
    w0 = w_ref[0]
    fa = f[:, :D]
    fb = f[:, D:]
    for h in range(2):
        wh = w0[:, h * DH:(h + 1) * DH]
        pa = jnp.dot(fa, wh, preferred_element_type=jnp.float32)
        pb = jnp.dot(fb, wh, preferred_element_type=jnp.float32)
        o_ref[h] = jnp.concatenate([pa, pb], axis=1)


def _relation_transform(f2, weight):
    bm = 5000
    nb = N2 // bm
    return pl.pallas_call(
        _mm_body,
        grid=(nb, R),
        in_specs=[
            pl.BlockSpec((bm, 2 * D), lambda b, r: (b, 0)),
            pl.BlockSpec((1, D, D), lambda b, r: (r, 0, 0)),
        ],
        out_specs=pl.BlockSpec((2, bm, D), lambda b, r, nb=nb: (0, r * nb + b, 0)),
        out_shape=jax.ShapeDtypeStruct((2, R * N2, D), jnp.float32),
    )(f2, weight)


BM5 = 1000


def _add_body(p_ref, o_ref):
    a = p_ref[0, 0] + p_ref[1, 0]
    b = p_ref[0, 1] + p_ref[1, 1]
    x = jnp.concatenate([a[:, :DH], b[:, :DH]], axis=1)
    y = jnp.concatenate([a[:, DH:], b[:, DH:]], axis=1)
    o_ref[...] = jnp.concatenate(
        [x.reshape(BM5, 1, D), y.reshape(BM5, 1, D)], axis=1
    ).reshape(2 * BM5, D)


def _combine_partials(partials):
    p5 = partials.reshape(NC, 2, N // 2, D)
    return pl.pallas_call(
        _add_body,
        grid=(N // (2 * BM5),),
        in_specs=[pl.BlockSpec((NC, 2, BM5, D), lambda b: (0, 0, b, 0))],
        out_specs=pl.BlockSpec((2 * BM5, D), lambda b: (b, 0)),
        out_shape=jax.ShapeDtypeStruct((N, D), jnp.float32),
    )(p5)


def _sc_body(hr_hbm, g_hbm, dst_hbm, norm_hbm, out_hbm,
             g_v, d_v, n_v, gbuf0, gbuf1, sbuf0, sbuf1, acc,
             gsem0, gsem1, ssem0, ssem1):
    cid = lax.axis_index("c")
    sid = lax.axis_index("s")
    w = cid * NS + sid

    gbufs = (gbuf0, gbuf1)
    sbufs = (sbuf0, sbuf1)
    gsems = (gsem0, gsem1)
    ssems = (ssem0, ssem1)

    pltpu.sync_copy(g_hbm.at[w], g_v)
    pltpu.sync_copy(dst_hbm.at[w], d_v)
    pltpu.sync_copy(norm_hbm.at[w], n_v)

    zeros16 = jnp.zeros((16,), jnp.float32)

    for h in range(2):
        tab = hr_hbm.at[h]

        @pl.loop(0, ZBLK)
        def _(e):
            for k in range(LANES_H):
                gbuf0[e, pl.ds(k * 16, 16)] = zeros16

        for j in range(ROWS_PT // ZBLK):
            pltpu.sync_copy(gbuf0.at[pl.ds(0, ZBLK)],
                            acc.at[pl.ds(sid * ROWS_PT + j * ZBLK, ZBLK)])

        @pl.when(sid == NS - 1)
        def _():
            pltpu.sync_copy(gbuf0.at[pl.ds(0, TAIL_ROWS)],
                            acc.at[pl.ds(N - TAIL_ROWS, TAIL_ROWS)])

        def start_gather(c, b):
            pltpu.async_copy(tab.at[g_v.at[c]], gbufs[b], gsems[b])

        def wait_gather(c, b):
            pltpu.make_async_copy(tab.at[g_v.at[c]], gbufs[b], gsems[b]).wait()

        def start_scatter(c, b):
            pltpu.async_copy(sbufs[b], acc.at[d_v.at[c]], ssems[b], add=True)

        def wait_scatter(c, b):
            pltpu.make_async_copy(sbufs[b], acc.at[d_v.at[c]], ssems[b]).wait()

        plsc.subcore_barrier()

        start_gather(0, 0)
        start_gather(1, 1)

        @pl.loop(0, NCHUNK, step=2)
        def _(c0):
            for b in range(2):
                c = c0 + b
                wait_gather(c, b)

                @pl.when(c0 > 0)
                def _():
                    wait_scatter(c - 2, b)

                def scale_group(off):
                    nv = n_v[c, pl.ds(off, 16)]
                    for i in range(16):
                        s = nv[i]
                        for k in range(LANES_H):
                            sl = pl.ds(k * 16, 16)
                            sbufs[b][off + i, sl] = gbufs[b][off + i, sl] * s

                @pl.loop(0, CHUNK // 16)
                def _(eg):
                    scale_group(eg * 16)

                @pl.when(c0 < NCHUNK - 2)
                def _():
                    start_gather(c + 2, b)

                start_scatter(c, b)

        wait_scatter(NCHUNK - 2, 0)
        wait_scatter(NCHUNK - 1, 1)

        plsc.subcore_barrier()
        pltpu.sync_copy(acc.at[pl.ds(sid * ROWS_PT, ROWS_PT)],
                        out_hbm.at[cid, h, pl.ds(sid * ROWS_PT, ROWS_PT)])

        @pl.when(sid == NS - 1)
        def _():
            pltpu.sync_copy(acc.at[pl.ds(N - TAIL_ROWS, TAIL_ROWS)],
                            out_hbm.at[cid, h, pl.ds(N - TAIL_ROWS, TAIL_ROWS)])


@functools.partial(
    pl.kernel,
    out_type=jax.ShapeDtypeStruct((NC, 2, N, DH), jnp.float32),
    mesh=plsc.VectorSubcoreMesh(
        core_axis_name="c", subcore_axis_name="s", num_cores=NC, num_subcores=NS
    ),
    compiler_params=pltpu.CompilerParams(use_tc_tiling_on_sc=False),
    scratch_types=[
        pltpu.VMEM((NCHUNK, CHUNK), jnp.int32),
        pltpu.VMEM((NCHUNK, CHUNK), jnp.int32),
        pltpu.VMEM((NCHUNK, CHUNK), jnp.float32),
        pltpu.VMEM((CHUNK, DH), jnp.float32),
        pltpu.VMEM((CHUNK, DH), jnp.float32),
        pltpu.VMEM((CHUNK, DH), jnp.float32),
        pltpu.VMEM((CHUNK, DH), jnp.float32),
        pltpu.VMEM_SHARED((N, DH), jnp.float32),
        pltpu.SemaphoreType.DMA,
        pltpu.SemaphoreType.DMA,
        pltpu.SemaphoreType.DMA,
        pltpu.SemaphoreType.DMA,
    ],
)
def _sc_edge_kernel(hr_hbm, g_hbm, dst_hbm, norm_hbm, out_hbm, *rest):
    _sc_body(hr_hbm, g_hbm, dst_hbm, norm_hbm, out_hbm, *rest)


def kernel(feature, edge_index, edge_type, norm, weight):
    f2 = feature.reshape(N2, 2 * D)
    hr = _relation_transform(f2, weight)
    src = edge_index[0]
    dst = edge_index[1]
    g = edge_type.astype(jnp.int32) * N + src
    pad = NW * EPT - E
    pad_idx = jnp.arange(pad, dtype=jnp.int32)
    gp = jnp.concatenate([g, pad_idx % (2 * R * N2)])
    dp = jnp.concatenate([dst, pad_idx % N])
    npad = jnp.concatenate([norm.reshape(E), jnp.zeros((pad,), jnp.float32)])
    partials = _sc_edge_kernel(
        hr.reshape(2, 2 * R * N2, DH),
        gp.reshape(NW, NCHUNK, CHUNK),
        dp.reshape(NW, NCHUNK, CHUNK),
        npad.reshape(NW, NCHUNK, CHUNK),
    )
    return _combine_partials(partials)


# speedup vs baseline: 2.7546x; 1.0424x over previous
import functools

import jax
import jax.numpy as jnp
from jax import lax
from jax.experimental import pallas as pl
from jax.experimental.pallas import tpu as pltpu
from jax.experimental.pallas import tpu_sc as plsc

N = 10000
E = 320000
D = 128
R = 8

NC = 2
NS = 16
NW = NC * NS
CHUNK = 128
NCHUNK = 81
NBUF = 3
EPT = CHUNK * NCHUNK
ROWS_PT = 624
TAIL_ROWS = N - 16 * ROWS_PT
ZBLK = 104
DH = D // 2
LANES_H = DH // 16
N2 = N // 2


def _mm_body(f_ref, w_ref, o_ref):
    f = f_ref[...]
    w0 = w_ref[0]
    fa = f[:, :D]
    fb = f[:, D:]
    for h in range(2):
        wh = w0[:, h * DH:(h + 1) * DH]
        pa = jnp.dot(fa, wh, preferred_element_type=jnp.float32)
        pb = jnp.dot(fb, wh, preferred_element_type=jnp.float32)
        o_ref[h] = jnp.concatenate([pa, pb], axis=1)


def _relation_transform(f2, weight):
    bm = 5000
    nb = N2 // bm
    return pl.pallas_call(
        _mm_body,
        grid=(nb, R),
        in_specs=[
            pl.BlockSpec((bm, 2 * D), lambda b, r: (b, 0)),
            pl.BlockSpec((1, D, D), lambda b, r: (r, 0, 0)),
        ],
        out_specs=pl.BlockSpec((2, bm, D), lambda b, r, nb=nb: (0, r * nb + b, 0)),
        out_shape=jax.ShapeDtypeStruct((2, R * N2, D), jnp.float32),
    )(f2, weight)


BM5 = 1000


def _add_body(p_ref, o_ref):
    a = p_ref[0, 0] + p_ref[1, 0]
    b = p_ref[0, 1] + p_ref[1, 1]
    x = jnp.concatenate([a[:, :DH], b[:, :DH]], axis=1)
    y = jnp.concatenate([a[:, DH:], b[:, DH:]], axis=1)
    o_ref[...] = jnp.concatenate(
        [x.reshape(BM5, 1, D), y.reshape(BM5, 1, D)], axis=1
    ).reshape(2 * BM5, D)


def _combine_partials(partials):
    p5 = partials.reshape(NC, 2, N // 2, D)
    return pl.pallas_call(
        _add_body,
        grid=(N // (2 * BM5),),
        in_specs=[pl.BlockSpec((NC, 2, BM5, D), lambda b: (0, 0, b, 0))],
        out_specs=pl.BlockSpec((2 * BM5, D), lambda b: (b, 0)),
        out_shape=jax.ShapeDtypeStruct((N, D), jnp.float32),
    )(p5)


def _sc_body(hr_hbm, g_hbm, dst_hbm, norm_hbm, out_hbm,
             g_v, d_v, n_v, gbuf0, gbuf1, gbuf2, sbuf0, sbuf1, sbuf2, acc,
             gsem0, gsem1, gsem2, ssem0, ssem1, ssem2):
    cid = lax.axis_index("c")
    sid = lax.axis_index("s")
    w = cid * NS + sid

    gbufs = (gbuf0, gbuf1, gbuf2)
    sbufs = (sbuf0, sbuf1, sbuf2)
    gsems = (gsem0, gsem1, gsem2)
    ssems = (ssem0, ssem1, ssem2)

    pltpu.sync_copy(g_hbm.at[w], g_v)
    pltpu.sync_copy(dst_hbm.at[w], d_v)
    pltpu.sync_copy(norm_hbm.at[w], n_v)

    zeros16 = jnp.zeros((16,), jnp.float32)

    for h in range(2):
        tab = hr_hbm.at[h]

        @pl.loop(0, ZBLK)
        def _(e):
            for k in range(LANES_H):
                gbuf0[e, pl.ds(k * 16, 16)] = zeros16

        for j in range(ROWS_PT // ZBLK):
            pltpu.sync_copy(gbuf0.at[pl.ds(0, ZBLK)],
                            acc.at[pl.ds(sid * ROWS_PT + j * ZBLK, ZBLK)])

        @pl.when(sid == NS - 1)
        def _():
            pltpu.sync_copy(gbuf0.at[pl.ds(0, TAIL_ROWS)],
                            acc.at[pl.ds(N - TAIL_ROWS, TAIL_ROWS)])

        def start_gather(c, b):
            pltpu.async_copy(tab.at[g_v.at[c]], gbufs[b], gsems[b])

        def wait_gather(c, b):
            pltpu.make_async_copy(tab.at[g_v.at[c]], gbufs[b], gsems[b]).wait()

        def start_scatter(c, b):
            pltpu.async_copy(sbufs[b], acc.at[d_v.at[c]], ssems[b], add=True)

        def wait_scatter(c, b):
            pltpu.make_async_copy(sbufs[b], acc.at[d_v.at[c]], ssems[b]).wait()

        plsc.subcore_barrier()

        for b in range(NBUF):
            start_gather(b, b)

        @pl.loop(0, NCHUNK, step=NBUF)
        def _(c0):
            for b in range(NBUF):
                c = c0 + b
                wait_gather(c, b)

                @pl.when(c0 > 0)
                def _():
                    wait_scatter(c - NBUF, b)

                def scale_group(off):
                    nv = n_v[c, pl.ds(off, 16)]
                    for i in range(16):
                        s = nv[i]
                        for k in range(LANES_H):
                            sl = pl.ds(k * 16, 16)
                            sbufs[b][off + i, sl] = gbufs[b][off + i, sl] * s

                @pl.loop(0, CHUNK // 16)
                def _(eg):
                    scale_group(eg * 16)

                @pl.when(c0 < NCHUNK - NBUF)
                def _():
                    start_gather(c + NBUF, b)

                start_scatter(c, b)

        for b in range(NBUF):
            wait_scatter(NCHUNK - NBUF + b, b)

        plsc.subcore_barrier()
        pltpu.sync_copy(acc.at[pl.ds(sid * ROWS_PT, ROWS_PT)],
                        out_hbm.at[cid, h, pl.ds(sid * ROWS_PT, ROWS_PT)])

        @pl.when(sid == NS - 1)
        def _():
            pltpu.sync_copy(acc.at[pl.ds(N - TAIL_ROWS, TAIL_ROWS)],
                            out_hbm.at[cid, h, pl.ds(N - TAIL_ROWS, TAIL_ROWS)])


@functools.partial(
    pl.kernel,
    out_type=jax.ShapeDtypeStruct((NC, 2, N, DH), jnp.float32),
    mesh=plsc.VectorSubcoreMesh(
        core_axis_name="c", subcore_axis_name="s", num_cores=NC, num_subcores=NS
    ),
    compiler_params=pltpu.CompilerParams(use_tc_tiling_on_sc=False),
    scratch_types=[
        pltpu.VMEM((NCHUNK, CHUNK), jnp.int32),
        pltpu.VMEM((NCHUNK, CHUNK), jnp.int32),
        pltpu.VMEM((NCHUNK, CHUNK), jnp.float32),
        pltpu.VMEM((CHUNK, DH), jnp.float32),
        pltpu.VMEM((CHUNK, DH), jnp.float32),
        pltpu.VMEM((CHUNK, DH), jnp.float32),
        pltpu.VMEM((CHUNK, DH), jnp.float32),
        pltpu.VMEM((CHUNK, DH), jnp.float32),
        pltpu.VMEM((CHUNK, DH), jnp.float32),
        pltpu.VMEM_SHARED((N, DH), jnp.float32),
        pltpu.SemaphoreType.DMA,
        pltpu.SemaphoreType.DMA,
        pltpu.SemaphoreType.DMA,
        pltpu.SemaphoreType.DMA,
        pltpu.SemaphoreType.DMA,
        pltpu.SemaphoreType.DMA,
    ],
)
def _sc_edge_kernel(hr_hbm, g_hbm, dst_hbm, norm_hbm, out_hbm, *rest):
    _sc_body(hr_hbm, g_hbm, dst_hbm, norm_hbm, out_hbm, *rest)


def kernel(feature, edge_index, edge_type, norm, weight):
    f2 = feature.reshape(N2, 2 * D)
    hr = _relation_transform(f2, weight)
    src = edge_index[0]
    dst = edge_index[1]
    g = edge_type.astype(jnp.int32) * N + src
    pad = NW * EPT - E
    pad_idx = jnp.arange(pad, dtype=jnp.int32)
    gp = jnp.concatenate([g, pad_idx % (2 * R * N2)])
    dp = jnp.concatenate([dst, pad_idx % N])
    npad = jnp.concatenate([norm.reshape(E), jnp.zeros((pad,), jnp.float32)])
    partials = _sc_edge_kernel(
        hr.reshape(2, 2 * R * N2, DH),
        gp.reshape(NW, NCHUNK, CHUNK),
        dp.reshape(NW, NCHUNK, CHUNK),
        npad.reshape(NW, NCHUNK, CHUNK),
    )
    return _combine_partials(partials)


# speedup vs baseline: 3.1478x; 1.1427x over previous
import functools

import jax
import jax.numpy as jnp
from jax import lax
from jax.experimental import pallas as pl
from jax.experimental.pallas import tpu as pltpu
from jax.experimental.pallas import tpu_sc as plsc

N = 10000
E = 320000
D = 128
R = 8

NC = 2
NS = 16
NW = NC * NS
CHUNK = 128
NCHUNK = 81
NBUF = 3
EPT = CHUNK * NCHUNK
ROWS_PT = 624
TAIL_ROWS = N - 16 * ROWS_PT
ZBLK = 104
DH = D // 2
LANES_H = DH // 16
N2 = N // 2


def _mm_body(f_ref, w_ref, o_ref):
    f = f_ref[...]
    flo = f[:N2]
    fhi = f[N2:]
    w0 = w_ref[0]
    for h in range(2):
        wh = w0[:, h * DH:(h + 1) * DH]
        lo = jnp.dot(flo, wh, preferred_element_type=jnp.float32)
        hi = jnp.dot(fhi, wh, preferred_element_type=jnp.float32)
        o_ref[h] = jnp.concatenate([lo, hi], axis=1)


def _relation_transform(feature, weight):
    return pl.pallas_call(
        _mm_body,
        grid=(R,),
        in_specs=[
            pl.BlockSpec((N, D), lambda r: (0, 0)),
            pl.BlockSpec((1, D, D), lambda r: (r, 0, 0)),
        ],
        out_specs=pl.BlockSpec((2, N2, D), lambda r: (0, r, 0)),
        out_shape=jax.ShapeDtypeStruct((2, R * N2, D), jnp.float32),
    )(feature, weight)


def _prep_body(e_ref, t_ref, g_ref, d_ref):
    src = e_ref[0]
    hi = (src >= N2).astype(jnp.int32)
    g_ref[...] = t_ref[...] * N + 2 * (src - N2 * hi) + hi
    d_ref[...] = e_ref[1]


def _edge_prep(edge_index, edge_type):
    return pl.pallas_call(
        _prep_body,
        out_shape=(
            jax.ShapeDtypeStruct((E,), jnp.int32),
            jax.ShapeDtypeStruct((E,), jnp.int32),
        ),
    )(edge_index, edge_type)


BM5 = 1000


def _add_body(p_ref, o_ref):
    a = p_ref[0, 0] + p_ref[1, 0]
    b = p_ref[0, 1] + p_ref[1, 1]
    x = jnp.concatenate([a[:, :DH], b[:, :DH]], axis=1)
    y = jnp.concatenate([a[:, DH:], b[:, DH:]], axis=1)
    o_ref[...] = jnp.concatenate(
        [x.reshape(BM5, 1, D), y.reshape(BM5, 1, D)], axis=1
    ).reshape(2 * BM5, D)


def _combine_partials(partials):
    p5 = partials.reshape(NC, 2, N // 2, D)
    return pl.pallas_call(
        _add_body,
        grid=(N // (2 * BM5),),
        in_specs=[pl.BlockSpec((NC, 2, BM5, D), lambda b: (0, 0, b, 0))],
        out_specs=pl.BlockSpec((2 * BM5, D), lambda b: (b, 0)),
        out_shape=jax.ShapeDtypeStruct((N, D), jnp.float32),
    )(p5)


def _sc_body(hr_hbm, g_hbm, dst_hbm, norm_hbm, out_hbm,
             g_v, d_v, n_v, gbuf0, gbuf1, gbuf2, sbuf0, sbuf1, sbuf2, acc,
             gsem0, gsem1, gsem2, ssem0, ssem1, ssem2):
    cid = lax.axis_index("c")
    sid = lax.axis_index("s")
    w = cid * NS + sid

    gbufs = (gbuf0, gbuf1, gbuf2)
    sbufs = (sbuf0, sbuf1, sbuf2)
    gsems = (gsem0, gsem1, gsem2)
    ssems = (ssem0, ssem1, ssem2)

    pltpu.sync_copy(g_hbm.at[w], g_v)
    pltpu.sync_copy(dst_hbm.at[w], d_v)
    pltpu.sync_copy(norm_hbm.at[w], n_v)

    zeros16 = jnp.zeros((16,), jnp.float32)

    for h in range(2):
        tab = hr_hbm.at[h]

        @pl.loop(0, ZBLK)
        def _(e):
            for k in range(LANES_H):
                gbuf0[e, pl.ds(k * 16, 16)] = zeros16

        for j in range(ROWS_PT // ZBLK):
            pltpu.sync_copy(gbuf0.at[pl.ds(0, ZBLK)],
                            acc.at[pl.ds(sid * ROWS_PT + j * ZBLK, ZBLK)])

        @pl.when(sid == NS - 1)
        def _():
            pltpu.sync_copy(gbuf0.at[pl.ds(0, TAIL_ROWS)],
                            acc.at[pl.ds(N - TAIL_ROWS, TAIL_ROWS)])

        def start_gather(c, b):
            pltpu.async_copy(tab.at[g_v.at[c]], gbufs[b], gsems[b])

        def wait_gather(c, b):
            pltpu.make_async_copy(tab.at[g_v.at[c]], gbufs[b], gsems[b]).wait()

        def start_scatter(c, b):
            pltpu.async_copy(sbufs[b], acc.at[d_v.at[c]], ssems[b], add=True)

        def wait_scatter(c, b):
            pltpu.make_async_copy(sbufs[b], acc.at[d_v.at[c]], ssems[b]).wait()

        plsc.subcore_barrier()

        for b in range(NBUF):
            start_gather(b, b)

        @pl.loop(0, NCHUNK, step=NBUF)
        def _(c0):
            for b in range(NBUF):
                c = c0 + b
                wait_gather(c, b)

                @pl.when(c0 > 0)
                def _():
                    wait_scatter(c - NBUF, b)

                def scale_group(off):
                    nv = n_v[c, pl.ds(off, 16)]
                    for i in range(16):
                        s = nv[i]
                        for k in range(LANES_H):
                            sl = pl.ds(k * 16, 16)
                            sbufs[b][off + i, sl] = gbufs[b][off + i, sl] * s

                @pl.loop(0, CHUNK // 16)
                def _(eg):
                    scale_group(eg * 16)

                @pl.when(c0 < NCHUNK - NBUF)
                def _():
                    start_gather(c + NBUF, b)

                start_scatter(c, b)

        for b in range(NBUF):
            wait_scatter(NCHUNK - NBUF + b, b)

        plsc.subcore_barrier()
        pltpu.sync_copy(acc.at[pl.ds(sid * ROWS_PT, ROWS_PT)],
                        out_hbm.at[cid, h, pl.ds(sid * ROWS_PT, ROWS_PT)])

        @pl.when(sid == NS - 1)
        def _():
            pltpu.sync_copy(acc.at[pl.ds(N - TAIL_ROWS, TAIL_ROWS)],
                            out_hbm.at[cid, h, pl.ds(N - TAIL_ROWS, TAIL_ROWS)])


@functools.partial(
    pl.kernel,
    out_type=jax.ShapeDtypeStruct((NC, 2, N, DH), jnp.float32),
    mesh=plsc.VectorSubcoreMesh(
        core_axis_name="c", subcore_axis_name="s", num_cores=NC, num_subcores=NS
    ),
    compiler_params=pltpu.CompilerParams(use_tc_tiling_on_sc=False),
    scratch_types=[
        pltpu.VMEM((NCHUNK, CHUNK), jnp.int32),
        pltpu.VMEM((NCHUNK, CHUNK), jnp.int32),
        pltpu.VMEM((NCHUNK, CHUNK), jnp.float32),
        pltpu.VMEM((CHUNK, DH), jnp.float32),
        pltpu.VMEM((CHUNK, DH), jnp.float32),
        pltpu.VMEM((CHUNK, DH), jnp.float32),
        pltpu.VMEM((CHUNK, DH), jnp.float32),
        pltpu.VMEM((CHUNK, DH), jnp.float32),
        pltpu.VMEM((CHUNK, DH), jnp.float32),
        pltpu.VMEM_SHARED((N, DH), jnp.float32),
        pltpu.SemaphoreType.DMA,
        pltpu.SemaphoreType.DMA,
        pltpu.SemaphoreType.DMA,
        pltpu.SemaphoreType.DMA,
        pltpu.SemaphoreType.DMA,
        pltpu.SemaphoreType.DMA,
    ],
)
def _sc_edge_kernel(hr_hbm, g_hbm, dst_hbm, norm_hbm, out_hbm, *rest):
    _sc_body(hr_hbm, g_hbm, dst_hbm, norm_hbm, out_hbm, *rest)


def kernel(feature, edge_index, edge_type, norm, weight):
    hr = _relation_transform(feature, weight)
    g, dst = _edge_prep(edge_index, edge_type)
    pad = NW * EPT - E
    pad_idx = jnp.arange(pad, dtype=jnp.int32)
    gp = jnp.concatenate([g, pad_idx % (2 * R * N2)])
    dp = jnp.concatenate([dst, pad_idx % N])
    npad = jnp.concatenate(
        [norm, jnp.zeros((pad, 1), jnp.float32)], axis=0)
    partials = _sc_edge_kernel(
        hr.reshape(2, 2 * R * N2, DH),
        gp.reshape(NW, NCHUNK, CHUNK),
        dp.reshape(NW, NCHUNK, CHUNK),
        npad.reshape(NW, NCHUNK, CHUNK),
    )
    return _combine_partials(partials)


# speedup vs baseline: 3.1755x; 1.0088x over previous
import functools

import jax
import jax.numpy as jnp
from jax import lax
from jax.experimental import pallas as pl
from jax.experimental.pallas import tpu as pltpu
from jax.experimental.pallas import tpu_sc as plsc

N = 10000
E = 320000
D = 128
R = 8

NC = 2
NS = 16
NW = NC * NS
CHUNK = 128
NCHUNK = 81
NBUF = 3
EPT = CHUNK * NCHUNK
ROWS_PT = 624
TAIL_ROWS = N - 16 * ROWS_PT
ZBLK = 104
DH = D // 2
LANES_H = DH // 16
N2 = N // 2


def _mm_body(f_ref, w_ref, o_ref):
    f = f_ref[...]
    flo = f[:N2]
    fhi = f[N2:]
    w0 = w_ref[0]
    for h in range(2):
        wh = w0[:, h * DH:(h + 1) * DH]
        lo = jnp.dot(flo, wh, preferred_element_type=jnp.float32)
        hi = jnp.dot(fhi, wh, preferred_element_type=jnp.float32)
        o_ref[h] = jnp.concatenate([lo, hi], axis=1)


def _relation_transform(feature, weight):
    return pl.pallas_call(
        _mm_body,
        grid=(R,),
        in_specs=[
            pl.BlockSpec((N, D), lambda r: (0, 0)),
            pl.BlockSpec((1, D, D), lambda r: (r, 0, 0)),
        ],
        out_specs=pl.BlockSpec((2, N2, D), lambda r: (0, r, 0)),
        out_shape=jax.ShapeDtypeStruct((2, R * N2, D), jnp.float32),
    )(feature, weight)


def _prep_body(e_ref, t_ref, g_ref, d_ref):
    src = e_ref[0]
    hi = (src >= N2).astype(jnp.int32)
    g_ref[...] = t_ref[...] * N + 2 * (src - N2 * hi) + hi
    d_ref[...] = e_ref[1]


def _edge_prep(edge_index, edge_type):
    return pl.pallas_call(
        _prep_body,
        out_shape=(
            jax.ShapeDtypeStruct((E,), jnp.int32),
            jax.ShapeDtypeStruct((E,), jnp.int32),
        ),
    )(edge_index, edge_type)


BM5 = 1000


def _add_body(p_ref, o_ref):
    a = p_ref[0, 0] + p_ref[1, 0]
    b = p_ref[0, 1] + p_ref[1, 1]
    x = jnp.concatenate([a[:, :DH], b[:, :DH]], axis=1)
    y = jnp.concatenate([a[:, DH:], b[:, DH:]], axis=1)
    o_ref[...] = jnp.concatenate(
        [x.reshape(BM5, 1, D), y.reshape(BM5, 1, D)], axis=1
    ).reshape(2 * BM5, D)


def _combine_partials(partials):
    p5 = partials.reshape(NC, 2, N // 2, D)
    return pl.pallas_call(
        _add_body,
        grid=(N // (2 * BM5),),
        in_specs=[pl.BlockSpec((NC, 2, BM5, D), lambda b: (0, 0, b, 0))],
        out_specs=pl.BlockSpec((2 * BM5, D), lambda b: (b, 0)),
        out_shape=jax.ShapeDtypeStruct((N, D), jnp.float32),
    )(p5)


def _sc_body(hr_hbm, g_hbm, dst_hbm, norm_hbm, out_hbm,
             g_v, d_v, n_v, gbuf0, gbuf1, gbuf2, sbuf0, sbuf1, sbuf2, zbuf,
             acc, gsem0, gsem1, gsem2, ssem0, ssem1, ssem2):
    cid = lax.axis_index("c")
    sid = lax.axis_index("s")
    w = cid * NS + sid

    gbufs = (gbuf0, gbuf1, gbuf2)
    sbufs = (sbuf0, sbuf1, sbuf2)
    gsems = (gsem0, gsem1, gsem2)
    ssems = (ssem0, ssem1, ssem2)
    tabs = (hr_hbm.at[0], hr_hbm.at[1])

    pltpu.sync_copy(g_hbm.at[w], g_v)
    pltpu.sync_copy(dst_hbm.at[w], d_v)
    pltpu.sync_copy(norm_hbm.at[w], n_v)

    zeros16 = jnp.zeros((16,), jnp.float32)

    @pl.loop(0, ZBLK)
    def _(e):
        for k in range(LANES_H):
            zbuf[e, pl.ds(k * 16, 16)] = zeros16

    def zero_acc():
        for j in range(ROWS_PT // ZBLK):
            pltpu.sync_copy(zbuf.at[pl.ds(0, ZBLK)],
                            acc.at[pl.ds(sid * ROWS_PT + j * ZBLK, ZBLK)])

        @pl.when(sid == NS - 1)
        def _():
            pltpu.sync_copy(zbuf.at[pl.ds(0, TAIL_ROWS)],
                            acc.at[pl.ds(N - TAIL_ROWS, TAIL_ROWS)])

    def start_gather(h, c, b):
        pltpu.async_copy(tabs[h].at[g_v.at[c]], gbufs[b], gsems[b])

    def wait_gather(h, c, b):
        pltpu.make_async_copy(tabs[h].at[g_v.at[c]], gbufs[b], gsems[b]).wait()

    def start_scatter(c, b):
        pltpu.async_copy(sbufs[b], acc.at[d_v.at[c]], ssems[b], add=True)

    def wait_scatter(c, b):
        pltpu.make_async_copy(sbufs[b], acc.at[d_v.at[c]], ssems[b]).wait()

    def main_loop(h):
        @pl.loop(0, NCHUNK, step=NBUF)
        def _(c0):
            for b in range(NBUF):
                c = c0 + b
                wait_gather(h, c, b)

                @pl.when(c0 > 0)
                def _():
                    wait_scatter(c - NBUF, b)

                def scale_group(off):
                    nv = n_v[c, pl.ds(off, 16)]
                    for i in range(16):
                        s = nv[i]
                        for k in range(LANES_H):
                            sl = pl.ds(k * 16, 16)
                            sbufs[b][off + i, sl] = gbufs[b][off + i, sl] * s

                @pl.loop(0, CHUNK // 16)
                def _(eg):
                    scale_group(eg * 16)

                @pl.when(c0 < NCHUNK - NBUF)
                def _():
                    start_gather(h, c + NBUF, b)

                start_scatter(c, b)

    def drain_scatters():
        for b in range(NBUF):
            wait_scatter(NCHUNK - NBUF + b, b)

    def copy_out(h):
        pltpu.sync_copy(acc.at[pl.ds(sid * ROWS_PT, ROWS_PT)],
                        out_hbm.at[cid, h, pl.ds(sid * ROWS_PT, ROWS_PT)])

        @pl.when(sid == NS - 1)
        def _():
            pltpu.sync_copy(acc.at[pl.ds(N - TAIL_ROWS, TAIL_ROWS)],
                            out_hbm.at[cid, h, pl.ds(N - TAIL_ROWS, TAIL_ROWS)])

    zero_acc()
    plsc.subcore_barrier()
    for b in range(NBUF):
        start_gather(0, b, b)
    main_loop(0)
    for b in range(NBUF):
        start_gather(1, b, b)
    drain_scatters()
    plsc.subcore_barrier()
    copy_out(0)

    zero_acc()
    plsc.subcore_barrier()
    main_loop(1)
    drain_scatters()
    plsc.subcore_barrier()
    copy_out(1)


@functools.partial(
    pl.kernel,
    out_type=jax.ShapeDtypeStruct((NC, 2, N, DH), jnp.float32),
    mesh=plsc.VectorSubcoreMesh(
        core_axis_name="c", subcore_axis_name="s", num_cores=NC, num_subcores=NS
    ),
    compiler_params=pltpu.CompilerParams(use_tc_tiling_on_sc=False),
    scratch_types=[
        pltpu.VMEM((NCHUNK, CHUNK), jnp.int32),
        pltpu.VMEM((NCHUNK, CHUNK), jnp.int32),
        pltpu.VMEM((NCHUNK, CHUNK), jnp.float32),
        pltpu.VMEM((CHUNK, DH), jnp.float32),
        pltpu.VMEM((CHUNK, DH), jnp.float32),
        pltpu.VMEM((CHUNK, DH), jnp.float32),
        pltpu.VMEM((CHUNK, DH), jnp.float32),
        pltpu.VMEM((CHUNK, DH), jnp.float32),
        pltpu.VMEM((CHUNK, DH), jnp.float32),
        pltpu.VMEM((ZBLK, DH), jnp.float32),
        pltpu.VMEM_SHARED((N, DH), jnp.float32),
        pltpu.SemaphoreType.DMA,
        pltpu.SemaphoreType.DMA,
        pltpu.SemaphoreType.DMA,
        pltpu.SemaphoreType.DMA,
        pltpu.SemaphoreType.DMA,
        pltpu.SemaphoreType.DMA,
    ],
)
def _sc_edge_kernel(hr_hbm, g_hbm, dst_hbm, norm_hbm, out_hbm, *rest):
    _sc_body(hr_hbm, g_hbm, dst_hbm, norm_hbm, out_hbm, *rest)


def kernel(feature, edge_index, edge_type, norm, weight):
    hr = _relation_transform(feature, weight)
    g, dst = _edge_prep(edge_index, edge_type)
    pad = NW * EPT - E
    pad_idx = jnp.arange(pad, dtype=jnp.int32)
    gp = jnp.concatenate([g, pad_idx % (2 * R * N2)])
    dp = jnp.concatenate([dst, pad_idx % N])
    npad = jnp.concatenate(
        [norm, jnp.zeros((pad, 1), jnp.float32)], axis=0)
    partials = _sc_edge_kernel(
        hr.reshape(2, 2 * R * N2, DH),
        gp.reshape(NW, NCHUNK, CHUNK),
        dp.reshape(NW, NCHUNK, CHUNK),
        npad.reshape(NW, NCHUNK, CHUNK),
    )
    return _combine_partials(partials)
